# baseline (speedup 1.0000x reference)
import functools

import jax
import jax.numpy as jnp
from jax import lax
from jax.experimental import pallas as pl
from jax.experimental.pallas import tpu as pltpu
from jax.experimental.pallas import tpu_sc as plsc

N = 10000
E = 320000
D_IN = 128
D_HID = 64
N_CLASSES = 47
K_HOPS = 8
C1 = 0.5
C2 = 0.5

NPAD = 10240
C = 256
CPW = 40
EPAD = 32 * CPW * C
RZT = NPAD // 16
RPT = NPAD // 32
NSC = 16


def _mesh():
    return plsc.VectorSubcoreMesh(
        core_axis_name="c", subcore_axis_name="s", num_cores=2, num_subcores=NSC
    )


NB = 1


def _make_prop(d):
    @functools.partial(
        pl.kernel,
        out_type=jax.ShapeDtypeStruct((2, NPAD, d), jnp.float32),
        mesh=_mesh(),
        scratch_types=[
            pltpu.VMEM((CPW, C), jnp.int32),
            pltpu.VMEM((CPW, C), jnp.int32),
            pltpu.VMEM((NB, C, d), jnp.float32),
            pltpu.VMEM_SHARED((NPAD, d), jnp.float32),
            pltpu.SemaphoreType.DMA((NB,)),
            pltpu.SemaphoreType.DMA((NB,)),
        ],
        compiler_params=pltpu.CompilerParams(use_tc_tiling_on_sc=False),
        name=f"vgcn_prop_{d}",
    )
    def prop(zs_hbm, src_hbm, dst_hbm, zer_hbm, acc_out,
             sidx, didx, gbufs, acc_sh, gsem, ssem):
        c = lax.axis_index("c")
        s = lax.axis_index("s")
        pltpu.sync_copy(zer_hbm, acc_sh.at[pl.ds(s * RZT, RZT)])
        plsc.subcore_barrier()
        w = c * NSC + s
        pltpu.sync_copy(src_hbm.at[w], sidx)
        pltpu.sync_copy(dst_hbm.at[w], didx)

        def chunk(j, carry):
            pltpu.async_copy(zs_hbm.at[sidx.at[j]], gbufs.at[0], gsem.at[0])
            pltpu.make_async_copy(zs_hbm.at[sidx.at[j]], gbufs.at[0],
                                  gsem.at[0]).wait()
            pltpu.sync_copy(gbufs.at[0], acc_sh.at[didx.at[j]], add=True)
            return carry

        lax.fori_loop(0, CPW, chunk, 0)

        plsc.subcore_barrier()
        pltpu.sync_copy(acc_sh.at[pl.ds(s * RZT, RZT)],
                        acc_out.at[c, pl.ds(s * RZT, RZT)])

    return prop


_prop64 = _make_prop(D_HID)
_prop48 = _make_prop(48)
_prop16 = _make_prop(16)


def _mlp(x, w, b, deg_acc, dout):
    bm = 1024
    kd = x.shape[1]

    def body(x_ref, w_ref, b_ref, dacc_ref, out_ref, outs_ref, dis_ref,
             d2_ref):
        deg = dacc_ref[0, :, :1] + dacc_ref[1, :, :1]
        dis = jnp.where(deg > 0.5, lax.rsqrt(deg), 0.0)
        acc = jnp.dot(x_ref[...], w_ref[...],
                      preferred_element_type=jnp.float32) + b_ref[...]
        out_ref[...] = acc
        outs_ref[...] = acc * dis
        dis_ref[...] = dis
        d2_ref[...] = dis * dis

    return pl.pallas_call(
        body,
        grid=(NPAD // bm,),
        in_specs=[
            pl.BlockSpec((bm, kd), lambda i: (i, 0)),
            pl.BlockSpec((kd, dout), lambda i: (0, 0)),
            pl.BlockSpec((1, dout), lambda i: (0, 0)),
            pl.BlockSpec((2, bm, 16), lambda i: (0, i, 0)),
        ],
        out_specs=[pl.BlockSpec((bm, dout), lambda i: (i, 0))] * 2
        + [pl.BlockSpec((bm, 1), lambda i: (i, 0))] * 2,
        out_shape=[jax.ShapeDtypeStruct((NPAD, dout), jnp.float32)] * 2
        + [jax.ShapeDtypeStruct((NPAD, 1), jnp.float32)] * 2,
    )(x, w, b.reshape(1, dout), deg_acc)


def _update(acc, a, b2d, dout):
    bm = 1024

    def body(acc_ref, a_ref, b_ref, o_ref):
        o_ref[...] = C1 * a_ref[...] + C2 * b_ref[...] * (acc_ref[0] + acc_ref[1])

    return pl.pallas_call(
        body,
        grid=(NPAD // bm,),
        in_specs=[
            pl.BlockSpec((2, bm, dout), lambda i: (0, i, 0)),
            pl.BlockSpec((bm, dout), lambda i: (i, 0)),
            pl.BlockSpec((bm, 1), lambda i: (i, 0)),
        ],
        out_specs=pl.BlockSpec((bm, dout), lambda i: (i, 0)),
        out_shape=jax.ShapeDtypeStruct((NPAD, dout), jnp.float32),
    )(acc, a, b2d)


def _block(prop, zs0, srcp, dstp, zer, ini, ini_s, dis2d, d22d, d):
    zs = zs0
    for _ in range(K_HOPS - 1):
        acc = prop(zs, srcp, dstp, zer)
        zs = _update(acc, ini_s, d22d, d)
    acc = prop(zs, srcp, dstp, zer)
    return _update(acc, ini, dis2d, d)


def kernel(features, edge_index, W1, b1, W2, b2):
    src = edge_index[0].astype(jnp.int32)
    dst = edge_index[1].astype(jnp.int32)
    srcp = jnp.concatenate(
        [src, jnp.full((EPAD - E,), N, jnp.int32)]).reshape(32, CPW, C)
    dstp = jnp.concatenate(
        [dst, jnp.full((EPAD - E,), NPAD - 1, jnp.int32)]).reshape(32, CPW, C)

    ones16 = jnp.broadcast_to(
        (jnp.arange(NPAD) < N).astype(jnp.float32)[:, None], (NPAD, 16))
    zer16 = jnp.zeros((RZT, 16), jnp.float32)
    zer64 = jnp.zeros((RZT, D_HID), jnp.float32)
    zer48 = jnp.zeros((RZT, 48), jnp.float32)

    deg_acc = _prop16(ones16, srcp, dstp, zer16)

    featp = jnp.pad(features, ((0, NPAD - N), (0, 0)))
    ini1, ini1_s, dis2d, d22d = _mlp(featp, W1, b1, deg_acc, D_HID)
    h = _block(_prop64, ini1_s, srcp, dstp, zer64, ini1, ini1_s, dis2d, d22d,
               D_HID)

    w2p = jnp.pad(W2, ((0, 0), (0, 1)))
    b2p = jnp.pad(b2, (0, 1))
    ini2, ini2_s, _, _ = _mlp(h, w2p, b2p, deg_acc, 48)
    out = _block(_prop48, ini2_s, srcp, dstp, zer48, ini2, ini2_s, dis2d, d22d,
                 48)
    return out[:N, :N_CLASSES]


# speedup vs baseline: 1.4629x; 1.4629x over previous
import functools

import jax
import jax.numpy as jnp
from jax import lax
from jax.experimental import pallas as pl
from jax.experimental.pallas import tpu as pltpu
from jax.experimental.pallas import tpu_sc as plsc

N = 10000
E = 320000
D_IN = 128
D_HID = 64
N_CLASSES = 47
K_HOPS = 8
C1 = 0.5
C2 = 0.5

NPAD = 10240
C = 128
CPW = 80
EPAD = 32 * CPW * C
RZT = NPAD // 16
RPT = NPAD // 32
NSC = 16


def _mesh():
    return plsc.VectorSubcoreMesh(
        core_axis_name="c", subcore_axis_name="s", num_cores=2, num_subcores=NSC
    )


NB = 1


def _make_prop(d):
    @functools.partial(
        pl.kernel,
        out_type=jax.ShapeDtypeStruct((2, NPAD, d), jnp.bfloat16),
        mesh=_mesh(),
        scratch_types=[
            pltpu.VMEM((CPW, C), jnp.int32),
            pltpu.VMEM((CPW, C), jnp.int32),
            pltpu.VMEM((NB, C, d), jnp.bfloat16),
            pltpu.VMEM_SHARED((NPAD, d), jnp.bfloat16),
            pltpu.SemaphoreType.DMA((NB,)),
            pltpu.SemaphoreType.DMA((NB,)),
        ],
        compiler_params=pltpu.CompilerParams(use_tc_tiling_on_sc=False),
        name=f"vgcn_prop_{d}",
    )
    def prop(zs_hbm, src_hbm, dst_hbm, zer_hbm, acc_out,
             sidx, didx, gbufs, acc_sh, gsem, ssem):
        c = lax.axis_index("c")
        s = lax.axis_index("s")
        pltpu.sync_copy(zer_hbm, acc_sh.at[pl.ds(s * RZT, RZT)])
        plsc.subcore_barrier()
        w = c * NSC + s
        pltpu.sync_copy(src_hbm.at[w], sidx)
        pltpu.sync_copy(dst_hbm.at[w], didx)

        def chunk(j, carry):
            pltpu.async_copy(zs_hbm.at[sidx.at[j]], gbufs.at[0], gsem.at[0])
            pltpu.make_async_copy(zs_hbm.at[sidx.at[j]], gbufs.at[0],
                                  gsem.at[0]).wait()
            pltpu.sync_copy(gbufs.at[0], acc_sh.at[didx.at[j]], add=True)
            return carry

        lax.fori_loop(0, CPW, chunk, 0)

        plsc.subcore_barrier()
        pltpu.sync_copy(acc_sh.at[pl.ds(s * RZT, RZT)],
                        acc_out.at[c, pl.ds(s * RZT, RZT)])

    return prop


_prop64 = _make_prop(D_HID)
_prop32 = _make_prop(32)


def _mlp(x, w, b, deg_acc, dout):
    bm = 1024
    kd = x.shape[1]

    def body(x_ref, w_ref, b_ref, dacc_ref, out_ref, outs_ref, zs0_ref,
             dis_ref, d2_ref):
        deg = (dacc_ref[0, :, :1] + dacc_ref[1, :, :1]).astype(jnp.float32)
        dis = jnp.where(deg > 0.5, lax.rsqrt(deg), 0.0)
        acc = jnp.dot(x_ref[...], w_ref[...],
                      preferred_element_type=jnp.float32) + b_ref[...]
        out_ref[...] = acc
        scaled = acc * dis
        outs_ref[...] = scaled
        zs0_ref[...] = scaled.astype(jnp.bfloat16)
        dis_ref[...] = dis
        d2_ref[...] = dis * dis

    return pl.pallas_call(
        body,
        grid=(NPAD // bm,),
        in_specs=[
            pl.BlockSpec((bm, kd), lambda i: (i, 0)),
            pl.BlockSpec((kd, dout), lambda i: (0, 0)),
            pl.BlockSpec((1, dout), lambda i: (0, 0)),
            pl.BlockSpec((2, bm, 32), lambda i: (0, i, 0)),
        ],
        out_specs=[pl.BlockSpec((bm, dout), lambda i: (i, 0))] * 3
        + [pl.BlockSpec((bm, 1), lambda i: (i, 0))] * 2,
        out_shape=[jax.ShapeDtypeStruct((NPAD, dout), jnp.float32)] * 2
        + [jax.ShapeDtypeStruct((NPAD, dout), jnp.bfloat16)]
        + [jax.ShapeDtypeStruct((NPAD, 1), jnp.float32)] * 2,
    )(x, w, b.reshape(1, dout), deg_acc)


def _update(acc, a, b2d, dout, out_dtype):
    bm = 1024

    def body(acc_ref, a_ref, b_ref, o_ref):
        s = (acc_ref[0] + acc_ref[1]).astype(jnp.float32)
        o_ref[...] = (C1 * a_ref[...] + C2 * b_ref[...] * s).astype(out_dtype)

    return pl.pallas_call(
        body,
        grid=(NPAD // bm,),
        in_specs=[
            pl.BlockSpec((2, bm, dout), lambda i: (0, i, 0)),
            pl.BlockSpec((bm, dout), lambda i: (i, 0)),
            pl.BlockSpec((bm, 1), lambda i: (i, 0)),
        ],
        out_specs=pl.BlockSpec((bm, dout), lambda i: (i, 0)),
        out_shape=jax.ShapeDtypeStruct((NPAD, dout), out_dtype),
    )(acc, a, b2d)


def _block(prop, zs0, srcp, dstp, zer, ini, ini_s, dis2d, d22d, d):
    zs = zs0
    for _ in range(K_HOPS - 1):
        acc = prop(zs, srcp, dstp, zer)
        zs = _update(acc, ini_s, d22d, d, jnp.bfloat16)
    acc = prop(zs, srcp, dstp, zer)
    return _update(acc, ini, dis2d, d, jnp.float32)


def kernel(features, edge_index, W1, b1, W2, b2):
    src = edge_index[0].astype(jnp.int32)
    dst = edge_index[1].astype(jnp.int32)
    srcp = jnp.concatenate(
        [src, jnp.full((EPAD - E,), N, jnp.int32)]).reshape(32, CPW, C)
    dstp = jnp.concatenate(
        [dst, jnp.full((EPAD - E,), NPAD - 1, jnp.int32)]).reshape(32, CPW, C)

    ones32 = jnp.broadcast_to(
        (jnp.arange(NPAD) < N).astype(jnp.bfloat16)[:, None], (NPAD, 32))
    zer32 = jnp.zeros((RZT, 32), jnp.bfloat16)
    zer64 = jnp.zeros((RZT, D_HID), jnp.bfloat16)

    deg_acc = _prop32(ones32, srcp, dstp, zer32)

    featp = jnp.pad(features, ((0, NPAD - N), (0, 0)))
    ini1, ini1_s, zs01, dis2d, d22d = _mlp(featp, W1, b1, deg_acc, D_HID)
    h = _block(_prop64, zs01, srcp, dstp, zer64, ini1, ini1_s, dis2d, d22d,
               D_HID)

    w2p = jnp.pad(W2, ((0, 0), (0, D_HID - N_CLASSES)))
    b2p = jnp.pad(b2, (0, D_HID - N_CLASSES))
    ini2, ini2_s, zs02, _, _ = _mlp(h, w2p, b2p, deg_acc, D_HID)
    out = _block(_prop64, zs02, srcp, dstp, zer64, ini2, ini2_s, dis2d, d22d,
                 D_HID)
    return out[:N, :N_CLASSES]


# speedup vs baseline: 1.5912x; 1.0877x over previous
import functools

import jax
import jax.numpy as jnp
from jax import lax
from jax.experimental import pallas as pl
from jax.experimental.pallas import tpu as pltpu
from jax.experimental.pallas import tpu_sc as plsc

N = 10000
E = 320000
D_IN = 128
D_HID = 64
N_CLASSES = 47
K_HOPS = 8
C1 = 0.5
C2 = 0.5

NPAD = 10240
C = 128
CPW = 80
EPAD = 32 * CPW * C
RZT = NPAD // 16
RPT = NPAD // 32
NSC = 16


def _mesh():
    return plsc.VectorSubcoreMesh(
        core_axis_name="c", subcore_axis_name="s", num_cores=2, num_subcores=NSC
    )


NB = 2


def _make_prop(d):
    @functools.partial(
        pl.kernel,
        out_type=jax.ShapeDtypeStruct((2, NPAD, d), jnp.bfloat16),
        mesh=_mesh(),
        scratch_types=[
            pltpu.VMEM((CPW, C), jnp.int32),
            pltpu.VMEM((CPW, C), jnp.int32),
            pltpu.VMEM((NB, C, d), jnp.bfloat16),
            pltpu.VMEM_SHARED((NPAD, d), jnp.bfloat16),
            pltpu.SemaphoreType.DMA((NB,)),
            pltpu.SemaphoreType.DMA((NB,)),
        ],
        compiler_params=pltpu.CompilerParams(use_tc_tiling_on_sc=False),
        name=f"vgcn_prop_{d}",
    )
    def prop(zs_hbm, src_hbm, dst_hbm, zer_hbm, acc_out,
             sidx, didx, gbufs, acc_sh, gsem, ssem):
        c = lax.axis_index("c")
        s = lax.axis_index("s")
        pltpu.sync_copy(zer_hbm, acc_sh.at[pl.ds(s * RZT, RZT)])
        plsc.subcore_barrier()
        w = c * NSC + s
        pltpu.sync_copy(src_hbm.at[w], sidx)
        pltpu.sync_copy(dst_hbm.at[w], didx)

        def g_start(j, b):
            pltpu.async_copy(zs_hbm.at[sidx.at[j]], gbufs.at[b], gsem.at[b])

        def g_wait(j, b):
            pltpu.make_async_copy(zs_hbm.at[sidx.at[j]], gbufs.at[b],
                                  gsem.at[b]).wait()

        g_start(0, 0)

        def chunk2(g, carry):
            j = g * 2
            g_wait(j, 0)
            g_start(j + 1, 1)
            pltpu.sync_copy(gbufs.at[0], acc_sh.at[didx.at[j]], add=True)
            g_wait(j + 1, 1)

            def more():
                g_start(j + 2, 0)

            pl.when(j + 2 < CPW)(more)
            pltpu.sync_copy(gbufs.at[1], acc_sh.at[didx.at[j + 1]], add=True)
            return carry

        lax.fori_loop(0, CPW // 2, chunk2, 0)

        plsc.subcore_barrier()
        pltpu.sync_copy(acc_sh.at[pl.ds(s * RZT, RZT)],
                        acc_out.at[c, pl.ds(s * RZT, RZT)])

    return prop


_prop64 = _make_prop(D_HID)
_prop32 = _make_prop(32)


def _mlp(x, w, b, deg_acc, dout):
    bm = 1024
    kd = x.shape[1]

    def body(x_ref, w_ref, b_ref, dacc_ref, out_ref, outs_ref, zs0_ref,
             dis_ref, d2_ref):
        deg = (dacc_ref[0, :, :1] + dacc_ref[1, :, :1]).astype(jnp.float32)
        dis = jnp.where(deg > 0.5, lax.rsqrt(deg), 0.0)
        acc = jnp.dot(x_ref[...], w_ref[...],
                      preferred_element_type=jnp.float32) + b_ref[...]
        out_ref[...] = acc
        scaled = acc * dis
        outs_ref[...] = scaled
        zs0_ref[...] = scaled.astype(jnp.bfloat16)
        dis_ref[...] = dis
        d2_ref[...] = dis * dis

    return pl.pallas_call(
        body,
        grid=(NPAD // bm,),
        in_specs=[
            pl.BlockSpec((bm, kd), lambda i: (i, 0)),
            pl.BlockSpec((kd, dout), lambda i: (0, 0)),
            pl.BlockSpec((1, dout), lambda i: (0, 0)),
            pl.BlockSpec((2, bm, 32), lambda i: (0, i, 0)),
        ],
        out_specs=[pl.BlockSpec((bm, dout), lambda i: (i, 0))] * 3
        + [pl.BlockSpec((bm, 1), lambda i: (i, 0))] * 2,
        out_shape=[jax.ShapeDtypeStruct((NPAD, dout), jnp.float32)] * 2
        + [jax.ShapeDtypeStruct((NPAD, dout), jnp.bfloat16)]
        + [jax.ShapeDtypeStruct((NPAD, 1), jnp.float32)] * 2,
    )(x, w, b.reshape(1, dout), deg_acc)


def _update(acc, a, b2d, dout, out_dtype):
    bm = 1024

    def body(acc_ref, a_ref, b_ref, o_ref):
        s = (acc_ref[0] + acc_ref[1]).astype(jnp.float32)
        o_ref[...] = (C1 * a_ref[...] + C2 * b_ref[...] * s).astype(out_dtype)

    return pl.pallas_call(
        body,
        grid=(NPAD // bm,),
        in_specs=[
            pl.BlockSpec((2, bm, dout), lambda i: (0, i, 0)),
            pl.BlockSpec((bm, dout), lambda i: (i, 0)),
            pl.BlockSpec((bm, 1), lambda i: (i, 0)),
        ],
        out_specs=pl.BlockSpec((bm, dout), lambda i: (i, 0)),
        out_shape=jax.ShapeDtypeStruct((NPAD, dout), out_dtype),
    )(acc, a, b2d)


def _block(prop, zs0, srcp, dstp, zer, ini, ini_s, dis2d, d22d, d):
    zs = zs0
    for _ in range(K_HOPS - 1):
        acc = prop(zs, srcp, dstp, zer)
        zs = _update(acc, ini_s, d22d, d, jnp.bfloat16)
    acc = prop(zs, srcp, dstp, zer)
    return _update(acc, ini, dis2d, d, jnp.float32)


def kernel(features, edge_index, W1, b1, W2, b2):
    src = edge_index[0].astype(jnp.int32)
    dst = edge_index[1].astype(jnp.int32)
    srcp = jnp.concatenate(
        [src, jnp.full((EPAD - E,), N, jnp.int32)]).reshape(32, CPW, C)
    dstp = jnp.concatenate(
        [dst, jnp.full((EPAD - E,), NPAD - 1, jnp.int32)]).reshape(32, CPW, C)

    ones32 = jnp.broadcast_to(
        (jnp.arange(NPAD) < N).astype(jnp.bfloat16)[:, None], (NPAD, 32))
    zer32 = jnp.zeros((RZT, 32), jnp.bfloat16)
    zer64 = jnp.zeros((RZT, D_HID), jnp.bfloat16)

    deg_acc = _prop32(ones32, srcp, dstp, zer32)

    featp = jnp.pad(features, ((0, NPAD - N), (0, 0)))
    ini1, ini1_s, zs01, dis2d, d22d = _mlp(featp, W1, b1, deg_acc, D_HID)
    h = _block(_prop64, zs01, srcp, dstp, zer64, ini1, ini1_s, dis2d, d22d,
               D_HID)

    w2p = jnp.pad(W2, ((0, 0), (0, D_HID - N_CLASSES)))
    b2p = jnp.pad(b2, (0, D_HID - N_CLASSES))
    ini2, ini2_s, zs02, _, _ = _mlp(h, w2p, b2p, deg_acc, D_HID)
    out = _block(_prop64, zs02, srcp, dstp, zer64, ini2, ini2_s, dis2d, d22d,
                 D_HID)
    return out[:N, :N_CLASSES]
